# baseline (speedup 1.0000x reference)
import functools

import jax
import jax.numpy as jnp
from jax import lax
from jax.experimental import pallas as pl
from jax.experimental.pallas import tpu as pltpu
from jax.experimental.pallas import tpu_sc as plsc

_D = 64
_DP = 128
_HS = 100
_NC = 2
_NS = 16
_NW = _NC * _NS
_NB = 4


def _build_gather(seq_per_worker, seq_len, npairs):
    nsteps = seq_per_worker // 2
    mesh = plsc.VectorSubcoreMesh(core_axis_name="c", subcore_axis_name="s")

    @functools.partial(
        pl.kernel,
        mesh=mesh,
        compiler_params=pltpu.CompilerParams(
            needs_layout_passes=False, use_tc_tiling_on_sc=False),
        out_type=jax.ShapeDtypeStruct((npairs * seq_len, _DP), jnp.float32),
        scratch_types=[
            pltpu.VMEM((4 * nsteps, _HS), jnp.int32),
        ] + [pltpu.VMEM((_HS, _D), jnp.float32)] * _NB
        + [pltpu.SemaphoreType.DMA] * (2 * _NB),
    )
    def k(idx_hbm, table_hbm, out_hbm, idx_v, *bufs_sems):
        bufs = bufs_sems[:_NB]
        gsems = bufs_sems[_NB:2 * _NB]
        ssems = bufs_sems[2 * _NB:]
        wid = lax.axis_index("s") * _NC + lax.axis_index("c")
        pair0 = wid * nsteps
        pltpu.sync_copy(idx_hbm.at[wid], idx_v)

        def dst(t, u):
            prow = (pair0 + t) * seq_len + (u % 2) * _HS
            return out_hbm.at[pl.ds(prow, _HS),
                              pl.ds((u // 2) * _D, _D)]

        def issue_gather(t, u):
            pltpu.async_copy(
                table_hbm.at[idx_v.at[4 * t + u]], bufs[u], gsems[u])

        def wait_gather(t, u):
            pltpu.make_async_copy(
                table_hbm.at[idx_v.at[4 * t + u]], bufs[u], gsems[u]).wait()

        def issue_scatter(t, u):
            pltpu.async_copy(bufs[u], dst(t, u), ssems[u])

        def wait_scatter(t, u):
            pltpu.make_async_copy(bufs[u], dst(t, u), ssems[u]).wait()

        def body(t, c):
            for u in range(_NB):
                @pl.when(t > 0)
                def _():
                    wait_scatter(t - 1, u)

                issue_gather(t, u)
            for u in range(_NB):
                wait_gather(t, u)
                issue_scatter(t, u)
            return c

        lax.fori_loop(0, nsteps, body, 0)
        for u in range(_NB):
            wait_scatter(nsteps - 1, u)

    return k


_QB = 4
_RB = _QB * 200


def _ln_half(x, g, b):
    m = jnp.mean(x, axis=1, keepdims=True)
    c = x - m
    v = jnp.mean(c * c, axis=1, keepdims=True)
    return c * lax.rsqrt(v + 1e-5) * g + b


def _ln_block(x_ref, pos_ref, g_ref, b_ref, o_ref):
    x = x_ref[...]
    pos = pos_ref[...]
    g = g_ref[...]
    b = b_ref[...]
    yl = _ln_half(x[:, :_D] + pos, g, b)
    yr = _ln_half(x[:, _D:] + pos, g, b)
    o_ref[...] = jnp.concatenate([yl, yr], axis=1)


def _ln_apply(x2, pos_t, gamma, beta):
    grid = x2.shape[0] // _RB
    return pl.pallas_call(
        _ln_block,
        grid=(grid,),
        in_specs=[
            pl.BlockSpec((_RB, _DP), lambda i: (i, 0)),
            pl.BlockSpec((_RB, _D), lambda i: (0, 0)),
            pl.BlockSpec((1, _D), lambda i: (0, 0)),
            pl.BlockSpec((1, _D), lambda i: (0, 0)),
        ],
        out_specs=pl.BlockSpec((_RB, _DP), lambda i: (i, 0)),
        out_shape=jax.ShapeDtypeStruct(x2.shape, jnp.float32),
    )(x2, pos_t, gamma, beta)


def kernel(token_ids, token_table, pos_table, gamma, beta):
    B, L = token_ids.shape
    V, D = token_table.shape
    assert D == _D and pos_table.shape == (L, D) and L == 2 * _HS
    assert B % (2 * _NW) == 0
    spw = B // _NW
    idx3 = token_ids.astype(jnp.int32).reshape(
        _NW, spw // 2, 2, 2, _HS).reshape(_NW, 2 * spw, _HS)
    x2 = _build_gather(spw, L, B // 2)(idx3, token_table.astype(jnp.float32))
    pos_t = jnp.tile(pos_table.astype(jnp.float32), (_QB, 1))
    y2 = _ln_apply(
        x2, pos_t,
        gamma.astype(jnp.float32).reshape(1, _D),
        beta.astype(jnp.float32).reshape(1, _D))
    return (y2.reshape(B // 2, L, 2, _D)
            .transpose(0, 2, 1, 3)
            .reshape(B, L, _D))


# speedup vs baseline: 1.5746x; 1.5746x over previous
import functools

import jax
import jax.numpy as jnp
from jax import lax
from jax.experimental import pallas as pl
from jax.experimental.pallas import tpu as pltpu
from jax.experimental.pallas import tpu_sc as plsc

_D = 64
_DP = 128
_CHUNK = 128
_NC = 2
_NS = 16
_NW = _NC * _NS


def _build_gather(nchunks, nrows):
    mesh = plsc.VectorSubcoreMesh(core_axis_name="c", subcore_axis_name="s")

    @functools.partial(
        pl.kernel,
        mesh=mesh,
        compiler_params=pltpu.CompilerParams(
            needs_layout_passes=False, use_tc_tiling_on_sc=False),
        out_type=jax.ShapeDtypeStruct((nrows, _DP), jnp.float32),
        scratch_types=[
            pltpu.VMEM((nchunks, _CHUNK), jnp.int32),
            pltpu.VMEM((_CHUNK, _D), jnp.float32),
            pltpu.VMEM((_CHUNK, _D), jnp.float32),
            pltpu.SemaphoreType.DMA,
            pltpu.SemaphoreType.DMA,
            pltpu.SemaphoreType.DMA,
            pltpu.SemaphoreType.DMA,
        ],
    )
    def k(idx_hbm, table_hbm, out_hbm, idx_v, ibuf0, ibuf1,
          gsem0, gsem1, ssem0, ssem1):
        wid = lax.axis_index("s") * _NC + lax.axis_index("c")
        row0 = wid * (nchunks * _CHUNK)
        pltpu.sync_copy(idx_hbm.at[wid], idx_v)
        ibufs = (ibuf0, ibuf1)
        gsems = (gsem0, gsem1)
        ssems = (ssem0, ssem1)

        def dst(j):
            return out_hbm.at[pl.ds(row0 + j * _CHUNK, _CHUNK), pl.ds(0, _D)]

        def issue_gather(j, b):
            pltpu.async_copy(table_hbm.at[idx_v.at[j]], ibufs[b], gsems[b])

        def wait_gather(j, b):
            pltpu.make_async_copy(
                table_hbm.at[idx_v.at[j]], ibufs[b], gsems[b]).wait()

        def issue_scatter(j, b):
            pltpu.async_copy(ibufs[b], dst(j), ssems[b])

        def wait_scatter(j, b):
            pltpu.make_async_copy(ibufs[b], dst(j), ssems[b]).wait()

        issue_gather(0, 0)

        def body(t, c):
            for u in range(2):
                j = 2 * t + u
                wait_gather(j, u)

                @pl.when(j + 1 < nchunks)
                def _():
                    @pl.when(j >= 1)
                    def _():
                        wait_scatter(j - 1, 1 - u)

                    issue_gather(j + 1, 1 - u)

                issue_scatter(j, u)
            return c

        lax.fori_loop(0, nchunks // 2, body, 0)
        wait_scatter(nchunks - 2, 0)
        wait_scatter(nchunks - 1, 1)

    return k


_RB = 3200


def _ln_block(x_ref, pos_ref, g_ref, b_ref, o_ref):
    x = x_ref[...][:, :_D] + pos_ref[...]
    m = jnp.mean(x, axis=1, keepdims=True)
    c = x - m
    v = jnp.mean(c * c, axis=1, keepdims=True)
    y = c * lax.rsqrt(v + 1e-5) * g_ref[...] + b_ref[...]
    o_ref[...] = y.reshape(_RB // 200, 200, _D)


def _ln_apply(x2, pos_t, gamma, beta, batch, seq_len):
    nrows = x2.shape[0]
    grid = nrows // _RB
    return pl.pallas_call(
        _ln_block,
        grid=(grid,),
        in_specs=[
            pl.BlockSpec((_RB, _DP), lambda i: (i, 0)),
            pl.BlockSpec((_RB, _D), lambda i: (0, 0)),
            pl.BlockSpec((1, _D), lambda i: (0, 0)),
            pl.BlockSpec((1, _D), lambda i: (0, 0)),
        ],
        out_specs=pl.BlockSpec(
            (_RB // 200, 200, _D), lambda i: (i, 0, 0)),
        out_shape=jax.ShapeDtypeStruct((batch, seq_len, _D), jnp.float32),
    )(x2, pos_t, gamma, beta)


def kernel(token_ids, token_table, pos_table, gamma, beta):
    B, L = token_ids.shape
    V, D = token_table.shape
    assert D == _D and pos_table.shape == (L, D)
    total = B * L
    assert total % (_NW * _CHUNK) == 0 and _RB % L == 0
    nchunks = total // (_NW * _CHUNK)
    idx3 = token_ids.astype(jnp.int32).reshape(_NW, nchunks, _CHUNK)
    x2 = _build_gather(nchunks, total)(idx3, token_table.astype(jnp.float32))
    pos_t = jnp.tile(pos_table.astype(jnp.float32), (_RB // L, 1))
    return _ln_apply(
        x2, pos_t,
        gamma.astype(jnp.float32).reshape(1, _D),
        beta.astype(jnp.float32).reshape(1, _D),
        B, L)


# speedup vs baseline: 1.6738x; 1.0630x over previous
import functools

import jax
import jax.numpy as jnp
from jax import lax
from jax.experimental import pallas as pl
from jax.experimental.pallas import tpu as pltpu
from jax.experimental.pallas import tpu_sc as plsc

_D = 64
_DP = 128
_CHUNK = 128
_NC = 2
_NS = 16
_NW = _NC * _NS
_NB = 4


def _build_gather(nchunks, nrows):
    mesh = plsc.VectorSubcoreMesh(core_axis_name="c", subcore_axis_name="s")

    @functools.partial(
        pl.kernel,
        mesh=mesh,
        compiler_params=pltpu.CompilerParams(
            needs_layout_passes=False, use_tc_tiling_on_sc=False),
        out_type=jax.ShapeDtypeStruct((nrows, _DP), jnp.float32),
        scratch_types=[
            pltpu.VMEM((nchunks, _CHUNK), jnp.int32),
        ] + [pltpu.VMEM((_CHUNK, _D), jnp.float32)] * _NB
        + [pltpu.SemaphoreType.DMA] * (2 * _NB),
    )
    def k(idx_hbm, table_hbm, out_hbm, idx_v, *bufs_sems):
        bufs = bufs_sems[:_NB]
        gsems = bufs_sems[_NB:2 * _NB]
        ssems = bufs_sems[2 * _NB:]
        wid = lax.axis_index("s") * _NC + lax.axis_index("c")
        row0 = wid * (nchunks * _CHUNK)
        pltpu.sync_copy(idx_hbm.at[wid], idx_v)

        def dst(j):
            return out_hbm.at[pl.ds(row0 + j * _CHUNK, _CHUNK), pl.ds(0, _D)]

        def issue_gather(j, u):
            pltpu.async_copy(table_hbm.at[idx_v.at[j]], bufs[u], gsems[u])

        def wait_gather(j, u):
            pltpu.make_async_copy(
                table_hbm.at[idx_v.at[j]], bufs[u], gsems[u]).wait()

        def issue_scatter(j, u):
            pltpu.async_copy(bufs[u], dst(j), ssems[u])

        def wait_scatter(j, u):
            pltpu.make_async_copy(bufs[u], dst(j), ssems[u]).wait()

        def body(t, c):
            for u in range(_NB):
                @pl.when(t > 0)
                def _():
                    wait_scatter(_NB * (t - 1) + u, u)

                issue_gather(_NB * t + u, u)
            for u in range(_NB):
                wait_gather(_NB * t + u, u)
                issue_scatter(_NB * t + u, u)
            return c

        lax.fori_loop(0, nchunks // _NB, body, 0)
        for u in range(_NB):
            wait_scatter(nchunks - _NB + u, u)

    return k


_RB = 3200


def _ln_block(x_ref, pos_ref, g_ref, b_ref, o_ref):
    x = x_ref[...][:, :_D] + pos_ref[...]
    m = jnp.mean(x, axis=1, keepdims=True)
    c = x - m
    v = jnp.mean(c * c, axis=1, keepdims=True)
    y = c * lax.rsqrt(v + 1e-5) * g_ref[...] + b_ref[...]
    o_ref[...] = y.reshape(_RB // 200, 200, _D)


def _ln_apply(x2, pos_t, gamma, beta, batch, seq_len):
    nrows = x2.shape[0]
    grid = nrows // _RB
    return pl.pallas_call(
        _ln_block,
        grid=(grid,),
        in_specs=[
            pl.BlockSpec((_RB, _DP), lambda i: (i, 0)),
            pl.BlockSpec((_RB, _D), lambda i: (0, 0)),
            pl.BlockSpec((1, _D), lambda i: (0, 0)),
            pl.BlockSpec((1, _D), lambda i: (0, 0)),
        ],
        out_specs=pl.BlockSpec(
            (_RB // 200, 200, _D), lambda i: (i, 0, 0)),
        out_shape=jax.ShapeDtypeStruct((batch, seq_len, _D), jnp.float32),
    )(x2, pos_t, gamma, beta)


def kernel(token_ids, token_table, pos_table, gamma, beta):
    B, L = token_ids.shape
    V, D = token_table.shape
    assert D == _D and pos_table.shape == (L, D)
    total = B * L
    assert total % (_NW * _CHUNK) == 0 and _RB % L == 0
    nchunks = total // (_NW * _CHUNK)
    idx3 = token_ids.astype(jnp.int32).reshape(_NW, nchunks, _CHUNK)
    x2 = _build_gather(nchunks, total)(idx3, token_table.astype(jnp.float32))
    pos_t = jnp.tile(pos_table.astype(jnp.float32), (_RB // L, 1))
    return _ln_apply(
        x2, pos_t,
        gamma.astype(jnp.float32).reshape(1, _D),
        beta.astype(jnp.float32).reshape(1, _D),
        B, L)
